# baseline (speedup 1.0000x reference)
import functools

import jax
import jax.numpy as jnp
from jax import lax
from jax.experimental import pallas as pl
from jax.experimental.pallas import tpu as pltpu
from jax.experimental.pallas import tpu_sc as plsc

_NC = 2
_NS = 16
_NW = _NC * _NS
_L = 16
_CHUNK = 128


def _score_kernel(B, D, n_chunks, b_per_w):
    mesh = plsc.VectorSubcoreMesh(core_axis_name="c", subcore_axis_name="s")

    @functools.partial(
        pl.kernel,
        out_type=jax.ShapeDtypeStruct((B,), jnp.float32),
        mesh=mesh,
        compiler_params=pltpu.CompilerParams(
            needs_layout_passes=False, use_tc_tiling_on_sc=False),
        scratch_types=[
            pltpu.VMEM((b_per_w,), jnp.int32),
            pltpu.VMEM((b_per_w,), jnp.int32),
            pltpu.VMEM((b_per_w,), jnp.int32),
            pltpu.VMEM((_CHUNK, 64), jnp.float32),
            pltpu.VMEM((_CHUNK, 64), jnp.float32),
            pltpu.VMEM((_CHUNK, 64), jnp.float32),
            pltpu.VMEM((_CHUNK, 64), jnp.float32),
            pltpu.VMEM((_CHUNK, 64), jnp.float32),
            pltpu.VMEM((_CHUNK, 64), jnp.float32),
            pltpu.VMEM((_CHUNK * _L,), jnp.float32),
            pltpu.VMEM((b_per_w,), jnp.float32),
            pltpu.SemaphoreType.DMA,
        ],
    )
    def k(e1_hbm, rel_hbm, e2_hbm, er_hbm, ei_hbm, rr_hbm, ri_hbm, out_hbm,
          e1_v, rel_v, e2_v, e1r, e1i, e2r, e2i, wr, wi, part, score_v, sem):
        wid = lax.axis_index("s") * _NC + lax.axis_index("c")
        base = wid * b_per_w
        pltpu.sync_copy(e1_hbm.at[pl.ds(base, b_per_w)], e1_v)
        pltpu.sync_copy(rel_hbm.at[pl.ds(base, b_per_w)], rel_v)
        pltpu.sync_copy(e2_hbm.at[pl.ds(base, b_per_w)], e2_v)

        @pl.loop(0, n_chunks)
        def chunk_loop(c):
            off = c * _CHUNK
            i1 = e1_v.at[pl.ds(off, _CHUNK)]
            i2 = e2_v.at[pl.ds(off, _CHUNK)]
            iw = rel_v.at[pl.ds(off, _CHUNK)]
            cps = [
                pltpu.async_copy(er_hbm.at[i1], e1r, sem),
                pltpu.async_copy(ei_hbm.at[i1], e1i, sem),
                pltpu.async_copy(er_hbm.at[i2], e2r, sem),
                pltpu.async_copy(ei_hbm.at[i2], e2i, sem),
                pltpu.async_copy(rr_hbm.at[iw], wr, sem),
                pltpu.async_copy(ri_hbm.at[iw], wi, sem),
            ]
            for cp in cps:
                cp.wait()

            @pl.loop(0, _CHUNK)
            def row_loop(r):
                acc = None
                for kk in range(D // _L):
                    sl = pl.ds(kk * _L, _L)
                    a_r = e1r[r, sl]
                    a_i = e1i[r, sl]
                    b_r = e2r[r, sl]
                    b_i = e2i[r, sl]
                    w_r = wr[r, sl]
                    w_i = wi[r, sl]
                    t1 = w_r * a_r - w_i * a_i
                    t2 = w_r * a_i + w_i * a_r
                    term = b_r * t1 + b_i * t2
                    acc = term if acc is None else acc + term
                part[pl.ds(r * _L, _L)] = acc

            @pl.loop(0, _CHUNK // _L)
            def red_loop(g):
                rowbase = g * (_L * _L) + lax.iota(jnp.int32, _L) * _L
                s = None
                for col in range(_L):
                    v = plsc.load_gather(part, [rowbase + col])
                    s = v if s is None else s + v
                score_v[pl.ds(off + g * _L, _L)] = s

        pltpu.sync_copy(score_v, out_hbm.at[pl.ds(base, b_per_w)])

    return k


def kernel(e1_idx, rel_idx, e2_idx, emb_e_real, emb_e_img,
           emb_rel_real, emb_rel_img):
    B = e1_idx.shape[0]
    D = emb_e_real.shape[1]
    b_per_w = B // _NW
    n_chunks = b_per_w // _CHUNK
    k = _score_kernel(B, D, n_chunks, b_per_w)
    return k(e1_idx.astype(jnp.int32), rel_idx.astype(jnp.int32),
             e2_idx.astype(jnp.int32), emb_e_real, emb_e_img,
             emb_rel_real, emb_rel_img)
